# baseline (speedup 1.0000x reference)
import functools

import jax
import jax.numpy as jnp
from jax import lax
from jax.experimental import pallas as pl
from jax.experimental.pallas import tpu as pltpu
from jax.experimental.pallas import tpu_sc as plsc

_D_MODEL = 1024
_D_HEAD = 64
_N_HEADS = 64
_N_TOK = 2048
_K = 8
_N_OUT = _N_TOK * _K
_BN = 256


def _mm_body(x_ref, w_ref, b_ref, o_ref):
    acc = (
        jnp.dot(
            x_ref[...].astype(jnp.bfloat16),
            w_ref[...],
            preferred_element_type=jnp.float32,
        )
        + b_ref[...]
    )
    for i in range(_BN // 128):
        o_ref[i] = acc[:, i * 128 : (i + 1) * 128]


def _dense_all_heads(x, wt, bf):
    m, k = x.shape
    n = wt.shape[1]
    return pl.pallas_call(
        _mm_body,
        grid=(n // _BN,),
        in_specs=[
            pl.BlockSpec((m, k), lambda j: (0, 0)),
            pl.BlockSpec((k, _BN), lambda j: (0, j)),
            pl.BlockSpec((1, _BN), lambda j: (0, j)),
        ],
        out_specs=pl.BlockSpec((_BN // 128, m, 128), lambda j: (j, 0, 0)),
        out_shape=jax.ShapeDtypeStruct((n // 128, m, 128), jnp.float32),
    )(x, wt, bf)


_info = plsc.get_sparse_core_info()
_NC, _NS, _L = _info.num_cores, _info.num_subcores, _info.num_lanes
_NW = _NC * _NS
_BPW = _N_OUT // _NW
_CH = 128
_NCH = _BPW // _CH

_sc_mesh = plsc.VectorSubcoreMesh(core_axis_name="c", subcore_axis_name="s")


@functools.partial(
    pl.kernel,
    mesh=_sc_mesh,
    compiler_params=pltpu.CompilerParams(use_tc_tiling_on_sc=True),
    out_type=jax.ShapeDtypeStruct((_N_OUT, 128), jnp.float32),
    scratch_types=[
        pltpu.VMEM((_BPW,), jnp.int32),
        pltpu.VMEM((_BPW, 128), jnp.float32),
        pltpu.SemaphoreType.DMA,
        pltpu.SemaphoreType.DMA,
    ],
)
def _sc_gather(table_hbm, sel_hbm, out_hbm, idx_v, buf_v, sem, wsem):
    wid = lax.axis_index("s") * _NC + lax.axis_index("c")
    base = wid * _BPW
    pltpu.sync_copy(sel_hbm.at[pl.ds(base, _BPW)], idx_v)
    lanes = lax.iota(jnp.int32, _L)
    gpc = _CH // _L
    copies = []
    for c in range(_NCH):
        for j in range(c * gpc, (c + 1) * gpc):
            sel = idx_v[pl.ds(j * _L, _L)]
            t = (base + j * _L + lanes) >> 3
            idx_v[pl.ds(j * _L, _L)] = ((sel >> 1) << 11) + t
        copies.append(
            pltpu.async_copy(
                table_hbm.at[idx_v.at[pl.ds(c * _CH, _CH)]],
                buf_v.at[pl.ds(c * _CH, _CH)],
                sem,
            )
        )
    wcopies = []
    for c in range(_NCH):
        copies[c].wait()
        wcopies.append(
            pltpu.async_copy(
                buf_v.at[pl.ds(c * _CH, _CH)],
                out_hbm.at[pl.ds(base + c * _CH, _CH)],
                wsem,
            )
        )
    for wc in wcopies:
        wc.wait()


_TB = 512


def _sel_body(rows_ref, sel_ref, o_ref):
    rows3 = rows_ref[...].reshape(_TB, _K, 128)
    par = (sel_ref[...] & 1)[..., None]
    o_ref[...] = jnp.where(
        par == 1, rows3[:, :, _D_HEAD:], rows3[:, :, : _D_HEAD]
    )


def _parity_select(rows, sel2):
    return pl.pallas_call(
        _sel_body,
        grid=(_N_TOK // _TB,),
        in_specs=[
            pl.BlockSpec((_TB * _K, 128), lambda j: (j, 0)),
            pl.BlockSpec((_TB, _K), lambda j: (j, 0)),
        ],
        out_specs=pl.BlockSpec((_TB, _K, _D_HEAD), lambda j: (j, 0, 0)),
        out_shape=jax.ShapeDtypeStruct((_N_TOK, _K, _D_HEAD), jnp.float32),
    )(rows, sel2)


@jax.jit
def kernel(tensor, head_selections, W, b):
    wt = (
        jnp.transpose(W.astype(jnp.bfloat16), (1, 0, 2))
        .reshape(_D_MODEL, _N_HEADS * _D_HEAD)
    )
    table = _dense_all_heads(tensor, wt, b.reshape(1, -1))
    sel2 = head_selections.astype(jnp.int32)
    rows = _sc_gather(
        table.reshape((_N_HEADS * _D_HEAD // 128) * _N_TOK, 128),
        sel2.reshape(-1),
    )
    return _parity_select(rows, sel2)


# speedup vs baseline: 1.0284x; 1.0284x over previous
import functools

import jax
import jax.numpy as jnp
from jax import lax
from jax.experimental import pallas as pl
from jax.experimental.pallas import tpu as pltpu
from jax.experimental.pallas import tpu_sc as plsc

_D_MODEL = 1024
_D_HEAD = 64
_N_HEADS = 64
_N_TOK = 2048
_K = 8
_N_OUT = _N_TOK * _K
_BN = 512


def _mm_body(x_ref, w_ref, b_ref, o_ref):
    acc = (
        jnp.dot(
            x_ref[...].astype(jnp.bfloat16),
            w_ref[...],
            preferred_element_type=jnp.float32,
        )
        + b_ref[...]
    )
    for i in range(_BN // 128):
        o_ref[i] = acc[:, i * 128 : (i + 1) * 128]


def _dense_all_heads(x, wt, bf):
    m, k = x.shape
    n = wt.shape[1]
    return pl.pallas_call(
        _mm_body,
        grid=(n // _BN,),
        in_specs=[
            pl.BlockSpec((m, k), lambda j: (0, 0)),
            pl.BlockSpec((k, _BN), lambda j: (0, j)),
            pl.BlockSpec((1, _BN), lambda j: (0, j)),
        ],
        out_specs=pl.BlockSpec((_BN // 128, m, 128), lambda j: (j, 0, 0)),
        out_shape=jax.ShapeDtypeStruct((n // 128, m, 128), jnp.float32),
    )(x, wt, bf)


_info = plsc.get_sparse_core_info()
_NC, _NS, _L = _info.num_cores, _info.num_subcores, _info.num_lanes
_NW = _NC * _NS
_BPW = _N_OUT // _NW
_CH = 128
_NCH = _BPW // _CH

_sc_mesh = plsc.VectorSubcoreMesh(core_axis_name="c", subcore_axis_name="s")


@functools.partial(
    pl.kernel,
    mesh=_sc_mesh,
    compiler_params=pltpu.CompilerParams(use_tc_tiling_on_sc=True),
    out_type=jax.ShapeDtypeStruct((_N_OUT, 128), jnp.float32),
    scratch_types=[
        pltpu.VMEM((_BPW,), jnp.int32),
        pltpu.VMEM((_BPW, 128), jnp.float32),
        pltpu.SemaphoreType.DMA,
        pltpu.SemaphoreType.DMA,
    ],
)
def _sc_gather(table_hbm, sel_hbm, out_hbm, idx_v, buf_v, sem, wsem):
    wid = lax.axis_index("s") * _NC + lax.axis_index("c")
    base = wid * _BPW
    pltpu.sync_copy(sel_hbm.at[pl.ds(base, _BPW)], idx_v)
    lanes = lax.iota(jnp.int32, _L)
    gpc = _CH // _L
    copies = []
    for c in range(_NCH):
        for j in range(c * gpc, (c + 1) * gpc):
            sel = idx_v[pl.ds(j * _L, _L)]
            t = (base + j * _L + lanes) >> 3
            idx_v[pl.ds(j * _L, _L)] = ((sel >> 1) << 11) + t
        copies.append(
            pltpu.async_copy(
                table_hbm.at[idx_v.at[pl.ds(c * _CH, _CH)]],
                buf_v.at[pl.ds(c * _CH, _CH)],
                sem,
            )
        )
    wcopies = []
    for c in range(_NCH):
        copies[c].wait()
        wcopies.append(
            pltpu.async_copy(
                buf_v.at[pl.ds(c * _CH, _CH)],
                out_hbm.at[pl.ds(base + c * _CH, _CH)],
                wsem,
            )
        )
    for wc in wcopies:
        wc.wait()


_TB = 512


def _sel_body(rows_ref, sel_ref, o_ref):
    rows3 = rows_ref[...].reshape(_TB, _K, 128)
    par = (sel_ref[...] & 1)[..., None]
    o_ref[...] = jnp.where(
        par == 1, rows3[:, :, _D_HEAD:], rows3[:, :, : _D_HEAD]
    )


def _parity_select(rows, sel2):
    return pl.pallas_call(
        _sel_body,
        grid=(_N_TOK // _TB,),
        in_specs=[
            pl.BlockSpec((_TB * _K, 128), lambda j: (j, 0)),
            pl.BlockSpec((_TB, _K), lambda j: (j, 0)),
        ],
        out_specs=pl.BlockSpec((_TB, _K, _D_HEAD), lambda j: (j, 0, 0)),
        out_shape=jax.ShapeDtypeStruct((_N_TOK, _K, _D_HEAD), jnp.float32),
    )(rows, sel2)


@jax.jit
def kernel(tensor, head_selections, W, b):
    wt = (
        jnp.transpose(W.astype(jnp.bfloat16), (1, 0, 2))
        .reshape(_D_MODEL, _N_HEADS * _D_HEAD)
    )
    table = _dense_all_heads(tensor, wt, b.reshape(1, -1))
    sel2 = head_selections.astype(jnp.int32)
    rows = _sc_gather(
        table.reshape((_N_HEADS * _D_HEAD // 128) * _N_TOK, 128),
        sel2.reshape(-1),
    )
    return _parity_select(rows, sel2)
